# baseline (speedup 1.0000x reference)
import functools
import math

import jax
import jax.numpy as jnp
from jax import lax
from jax.experimental import pallas as pl
from jax.experimental.pallas import tpu as pltpu
from jax.experimental.pallas import tpu_sc as plsc

_NC = 2
_NS = 16
_NW = _NC * _NS
_L = 16

_BB = 128
_RSTRIDE = 128


def _pos_signal(length, channels, min_timescale=1.0, max_timescale=10000.0):
    num_timescales = channels // 2
    log_timescale_increment = math.log(
        float(max_timescale) / float(min_timescale)) / (float(num_timescales) - 1.0)
    position = jnp.arange(0, length, dtype=jnp.float32)
    inv_timescales = jnp.exp(
        jnp.arange(0, num_timescales, dtype=jnp.float32)
        * (-log_timescale_increment)) * min_timescale
    scaled_time = position[:, None] * inv_timescales[None, :]
    return jnp.concatenate([jnp.sin(scaled_time), jnp.cos(scaled_time)], axis=1)


def _make_sc_kernel(batch, seq_len, dim, scale):
    n_i = dim // 8
    mesh = plsc.VectorSubcoreMesh(core_axis_name="c", subcore_axis_name="s")

    @functools.partial(
        pl.kernel,
        out_type=jax.ShapeDtypeStruct((seq_len, n_i, _NW, 8, 128), jnp.float32),
        mesh=mesh,
        compiler_params=pltpu.CompilerParams(
            use_tc_tiling_on_sc=False, needs_layout_passes=False),
        scratch_types=[
            pltpu.VMEM((seq_len, _BB), jnp.int32),
            pltpu.VMEM((2, _BB, 64), jnp.float32),
            pltpu.VMEM((2, n_i, 8, 128), jnp.float32),
            pltpu.VMEM((2, dim, _L), jnp.float32),
            pltpu.SemaphoreType.DMA,
            pltpu.SemaphoreType.DMA,
            pltpu.SemaphoreType.DMA,
            pltpu.SemaphoreType.DMA,
        ],
    )
    def emb_kernel(table_hbm, idxt_hbm, sigb_hbm, out_hbm,
                   idx_v, rows_v, outt_v, sigl_v, g0, g1, o0, o1):
        cid = lax.axis_index("c")
        sid = lax.axis_index("s")
        wid = sid * _NC + cid
        gsems = (g0, g1)
        osems = (o0, o1)
        pltpu.sync_copy(idxt_hbm.at[:, pl.ds(wid * _BB, _BB)], idx_v)

        iota = lax.iota(jnp.int32, _L)

        def fire(l, b):
            pltpu.async_copy(
                table_hbm.at[idx_v.at[l]],
                rows_v.at[b],
                gsems[b],
            )
            pltpu.async_copy(sigb_hbm.at[l], sigl_v.at[b], gsems[b])

        def drain(l, b):
            pltpu.make_async_copy(
                table_hbm.at[idx_v.at[l]],
                rows_v.at[b],
                gsems[b],
            ).wait()
            pltpu.make_async_copy(sigb_hbm.at[l], sigl_v.at[b], gsems[b]).wait()

        def compute(l, b):
            for k in range(_BB // _L):
                rowidx = iota + k * _L
                for d in range(dim):
                    dvec = jnp.broadcast_to(jnp.int32(d), (_L,))
                    vals = plsc.load_gather(rows_v.at[b], [rowidx, dvec])
                    res = vals * scale + sigl_v[b, d, :]
                    outt_v[b, d // 8, d % 8, pl.ds(k * _L, _L)] = res

        def out_dma(l, b, make_only):
            mk = pltpu.make_async_copy if make_only else pltpu.async_copy
            for i in range(n_i):
                d = mk(outt_v.at[b].at[i], out_hbm.at[l, i, wid], osems[b])
                if make_only:
                    d.wait()

        fire(0, 0)

        @pl.loop(0, seq_len, step=2)
        def _lloop(l0):
            for b in range(2):
                l = l0 + b
                other = 1 - b

                @pl.when(l + 1 < seq_len)
                def _next():
                    @pl.when(l >= 1)
                    def _wait_out():
                        out_dma(l - 1, other, True)
                    fire(l + 1, other)

                drain(l, b)
                compute(l, b)
                out_dma(l, b, False)

        out_dma(seq_len - 2, 0, True)
        out_dma(seq_len - 1, 1, True)

    return emb_kernel


def kernel(x, table):
    batch, seq_len = x.shape
    num_emb, dim = table.shape
    scale = float(dim) ** 0.5
    sig = _pos_signal(seq_len, dim)
    sigb = jnp.broadcast_to(sig[:, :, None], (seq_len, dim, _L))
    xt = x.T
    tab2 = table
    sc = _make_sc_kernel(batch, seq_len, dim, scale)
    out5d = sc(tab2, xt, sigb)
    out = out5d.transpose(2, 4, 0, 1, 3).reshape(batch, seq_len, dim)
    return out


# speedup vs baseline: 1.3647x; 1.3647x over previous
import functools
import math

import jax
import jax.numpy as jnp
from jax import lax
from jax.experimental import pallas as pl
from jax.experimental.pallas import tpu as pltpu
from jax.experimental.pallas import tpu_sc as plsc

_NC = 2
_NS = 16
_NW = _NC * _NS
_L = 16

_BB = 128
_GB = 4


def _pos_signal(length, channels, min_timescale=1.0, max_timescale=10000.0):
    num_timescales = channels // 2
    log_timescale_increment = math.log(
        float(max_timescale) / float(min_timescale)) / (float(num_timescales) - 1.0)
    position = jnp.arange(0, length, dtype=jnp.float32)
    inv_timescales = jnp.exp(
        jnp.arange(0, num_timescales, dtype=jnp.float32)
        * (-log_timescale_increment)) * min_timescale
    scaled_time = position[:, None] * inv_timescales[None, :]
    return jnp.concatenate([jnp.sin(scaled_time), jnp.cos(scaled_time)], axis=1)


def _make_sc_kernel(batch, seq_len, dim, scale):
    n_i = dim // 8
    mesh = plsc.VectorSubcoreMesh(core_axis_name="c", subcore_axis_name="s")

    @functools.partial(
        pl.kernel,
        out_type=jax.ShapeDtypeStruct((seq_len, n_i, _NW, 8, 128), jnp.float32),
        mesh=mesh,
        compiler_params=pltpu.CompilerParams(
            use_tc_tiling_on_sc=False, needs_layout_passes=False),
        scratch_types=[
            pltpu.VMEM((seq_len, _BB), jnp.int32),
            pltpu.VMEM((_GB, _BB, 64), jnp.float32),
            pltpu.VMEM((_BB, 65), jnp.float32),
            pltpu.VMEM((2, n_i, 8, 128), jnp.float32),
            pltpu.VMEM((_GB, dim, _L), jnp.float32),
            pltpu.SemaphoreType.DMA,
            pltpu.SemaphoreType.DMA,
            pltpu.SemaphoreType.DMA,
            pltpu.SemaphoreType.DMA,
            pltpu.SemaphoreType.DMA,
            pltpu.SemaphoreType.DMA,
        ],
    )
    def emb_kernel(table_hbm, idxt_hbm, sigb_hbm, out_hbm,
                   idx_v, gbuf_v, rows_v, outt_v, sigl_v,
                   g0, g1, g2, g3, o0, o1):
        cid = lax.axis_index("c")
        sid = lax.axis_index("s")
        wid = sid * _NC + cid
        gsems = (g0, g1, g2, g3)
        osems = (o0, o1)
        pltpu.sync_copy(idxt_hbm.at[:, pl.ds(wid * _BB, _BB)], idx_v)

        iota = lax.iota(jnp.int32, _L)

        def fire(l, gb):
            pltpu.async_copy(table_hbm.at[idx_v.at[l]], gbuf_v.at[gb], gsems[gb])
            pltpu.async_copy(sigb_hbm.at[l], sigl_v.at[gb], gsems[gb])

        def drain(l, gb):
            pltpu.make_async_copy(
                table_hbm.at[idx_v.at[l]], gbuf_v.at[gb], gsems[gb]).wait()
            pltpu.make_async_copy(
                sigb_hbm.at[l], sigl_v.at[gb], gsems[gb]).wait()

        def restride(gb):
            @pl.loop(0, _BB, unroll=4)
            def _r(j):
                for d in range(dim // _L):
                    sl = pl.ds(d * _L, _L)
                    rows_v[j, sl] = gbuf_v[gb, j, sl]

        def compute(gb, ob):
            @pl.loop(0, _BB // _L)
            def _k(k):
                rowidx = iota + k * _L
                for d in range(dim):
                    dvec = jnp.broadcast_to(jnp.int32(d), (_L,))
                    vals = plsc.load_gather(rows_v, [rowidx, dvec])
                    res = vals * scale + sigl_v[gb, d, :]
                    outt_v[ob, d // 8, d % 8, pl.ds(k * _L, _L)] = res

        def out_dma(l, ob, make_only):
            mk = pltpu.make_async_copy if make_only else pltpu.async_copy
            for i in range(n_i):
                d = mk(outt_v.at[ob].at[i], out_hbm.at[l, i, wid], osems[ob])
                if make_only:
                    d.wait()

        fire(0, 0)
        fire(1, 1)
        fire(2, 2)

        @pl.loop(0, seq_len, step=_GB)
        def _lloop(l0):
            for b in range(_GB):
                l = l0 + b
                gb = b
                ob = b % 2

                @pl.when(l + 3 < seq_len)
                def _next():
                    fire(l + 3, (gb + 3) % _GB)

                drain(l, gb)
                restride(gb)

                @pl.when(l >= 2)
                def _wait_out():
                    out_dma(l - 2, ob, True)

                compute(gb, ob)
                out_dma(l, ob, False)

        out_dma(seq_len - 2, (seq_len - 2) % 2, True)
        out_dma(seq_len - 1, (seq_len - 1) % 2, True)

    return emb_kernel


def kernel(x, table):
    batch, seq_len = x.shape
    num_emb, dim = table.shape
    scale = float(dim) ** 0.5
    sig = _pos_signal(seq_len, dim)
    sigb = jnp.broadcast_to(sig[:, :, None], (seq_len, dim, _L))
    xt = x.T
    sc = _make_sc_kernel(batch, seq_len, dim, scale)
    out5d = sc(table, xt, sigb)
    out = out5d.transpose(2, 4, 0, 1, 3).reshape(batch, seq_len, dim)
    return out


# speedup vs baseline: 1.4779x; 1.0830x over previous
import functools
import math

import jax
import jax.numpy as jnp
from jax import lax
from jax.experimental import pallas as pl
from jax.experimental.pallas import tpu as pltpu
from jax.experimental.pallas import tpu_sc as plsc

_NC = 2
_NS = 16
_NW = _NC * _NS
_L = 16

_BB = 128
_GB = 4


def _pos_signal(length, channels, min_timescale=1.0, max_timescale=10000.0):
    num_timescales = channels // 2
    log_timescale_increment = math.log(
        float(max_timescale) / float(min_timescale)) / (float(num_timescales) - 1.0)
    position = jnp.arange(0, length, dtype=jnp.float32)
    inv_timescales = jnp.exp(
        jnp.arange(0, num_timescales, dtype=jnp.float32)
        * (-log_timescale_increment)) * min_timescale
    scaled_time = position[:, None] * inv_timescales[None, :]
    return jnp.concatenate([jnp.sin(scaled_time), jnp.cos(scaled_time)], axis=1)


def _make_sc_kernel(batch, seq_len, dim, scale):
    n_i = dim // 8
    mesh = plsc.VectorSubcoreMesh(core_axis_name="c", subcore_axis_name="s")

    @functools.partial(
        pl.kernel,
        out_type=jax.ShapeDtypeStruct((seq_len, n_i, _NW, 8, 128), jnp.float32),
        mesh=mesh,
        compiler_params=pltpu.CompilerParams(
            use_tc_tiling_on_sc=False, needs_layout_passes=False),
        scratch_types=[
            pltpu.VMEM((seq_len, _BB), jnp.int32),
            pltpu.VMEM((_GB, _BB, 64), jnp.float32),
            pltpu.VMEM((_BB * 65,), jnp.float32),
            pltpu.VMEM((2, n_i, 8, 128), jnp.float32),
            pltpu.VMEM((_GB, dim, _L), jnp.float32),
            pltpu.SemaphoreType.DMA,
            pltpu.SemaphoreType.DMA,
            pltpu.SemaphoreType.DMA,
            pltpu.SemaphoreType.DMA,
            pltpu.SemaphoreType.DMA,
            pltpu.SemaphoreType.DMA,
        ],
    )
    def emb_kernel(table_hbm, idxt_hbm, sigb_hbm, out_hbm,
                   idx_v, gbuf_v, rows_v, outt_v, sigl_v,
                   g0, g1, g2, g3, o0, o1):
        cid = lax.axis_index("c")
        sid = lax.axis_index("s")
        wid = sid * _NC + cid
        gsems = (g0, g1, g2, g3)
        osems = (o0, o1)
        pltpu.sync_copy(idxt_hbm.at[:, pl.ds(wid * _BB, _BB)], idx_v)

        iota = lax.iota(jnp.int32, _L)

        def fire(l, gb):
            pltpu.async_copy(table_hbm.at[idx_v.at[l]], gbuf_v.at[gb], gsems[gb])
            pltpu.async_copy(sigb_hbm.at[l], sigl_v.at[gb], gsems[gb])

        def drain(l, gb):
            pltpu.make_async_copy(
                table_hbm.at[idx_v.at[l]], gbuf_v.at[gb], gsems[gb]).wait()
            pltpu.make_async_copy(
                sigb_hbm.at[l], sigl_v.at[gb], gsems[gb]).wait()

        def restride(gb):
            @pl.loop(0, _BB, unroll=4)
            def _r(j):
                for d in range(dim // _L):
                    rows_v[pl.ds(j * 65 + d * _L, _L)] = gbuf_v[gb, j, pl.ds(d * _L, _L)]

        def compute(gb, ob):
            @pl.loop(0, _BB // _L)
            def _k(k):
                base = (iota + k * _L) * 65
                for d in range(dim):
                    vals = plsc.load_gather(rows_v, [base + d])
                    res = vals * scale + sigl_v[gb, d, :]
                    outt_v[ob, d // 8, d % 8, pl.ds(k * _L, _L)] = res

        def out_dma(l, ob, make_only):
            mk = pltpu.make_async_copy if make_only else pltpu.async_copy
            for i in range(n_i):
                d = mk(outt_v.at[ob].at[i], out_hbm.at[l, i, wid], osems[ob])
                if make_only:
                    d.wait()

        fire(0, 0)
        fire(1, 1)
        fire(2, 2)

        @pl.loop(0, seq_len, step=_GB)
        def _lloop(l0):
            for b in range(_GB):
                l = l0 + b
                gb = b
                ob = b % 2

                @pl.when(l + 3 < seq_len)
                def _next():
                    fire(l + 3, (gb + 3) % _GB)

                drain(l, gb)
                restride(gb)

                @pl.when(l >= 2)
                def _wait_out():
                    out_dma(l - 2, ob, True)

                compute(gb, ob)
                out_dma(l, ob, False)

        out_dma(seq_len - 2, (seq_len - 2) % 2, True)
        out_dma(seq_len - 1, (seq_len - 1) % 2, True)

    return emb_kernel


def kernel(x, table):
    batch, seq_len = x.shape
    num_emb, dim = table.shape
    scale = float(dim) ** 0.5
    sig = _pos_signal(seq_len, dim)
    sigb = jnp.broadcast_to(sig[:, :, None], (seq_len, dim, _L))
    xt = x.T
    sc = _make_sc_kernel(batch, seq_len, dim, scale)
    out5d = sc(table, xt, sigb)
    out = out5d.transpose(2, 4, 0, 1, 3).reshape(batch, seq_len, dim)
    return out


# speedup vs baseline: 1.8036x; 1.2204x over previous
import functools
import math

import jax
import jax.numpy as jnp
from jax import lax
from jax.experimental import pallas as pl
from jax.experimental.pallas import tpu as pltpu
from jax.experimental.pallas import tpu_sc as plsc

_NC = 2
_NS = 16
_NW = _NC * _NS
_LANES = 16

_CHUNK = 400
_GPIECE = 80


def _pos_signal(length, channels, min_timescale=1.0, max_timescale=10000.0):
    num_timescales = channels // 2
    log_timescale_increment = math.log(
        float(max_timescale) / float(min_timescale)) / (float(num_timescales) - 1.0)
    position = jnp.arange(0, length, dtype=jnp.float32)
    inv_timescales = jnp.exp(
        jnp.arange(0, num_timescales, dtype=jnp.float32)
        * (-log_timescale_increment)) * min_timescale
    scaled_time = position[:, None] * inv_timescales[None, :]
    return jnp.concatenate([jnp.sin(scaled_time), jnp.cos(scaled_time)], axis=1)


def _make_sc_kernel(n_rows, seq_len, dim, scale):
    per_w = n_rows // _NW
    n_chunks = per_w // _CHUNK
    n_pieces = _CHUNK // _GPIECE
    half = _CHUNK // 2
    mesh = plsc.VectorSubcoreMesh(core_axis_name="c", subcore_axis_name="s")

    @functools.partial(
        pl.kernel,
        out_type=jax.ShapeDtypeStruct((n_rows, dim), jnp.float32),
        mesh=mesh,
        compiler_params=pltpu.CompilerParams(use_tc_tiling_on_sc=False),
        scratch_types=[
            pltpu.VMEM((seq_len, dim), jnp.float32),
            pltpu.VMEM((per_w,), jnp.int32),
            pltpu.VMEM((2, _CHUNK, dim), jnp.float32),
            pltpu.SemaphoreType.DMA,
            pltpu.SemaphoreType.DMA,
            pltpu.SemaphoreType.DMA,
            pltpu.SemaphoreType.DMA,
        ],
    )
    def emb_kernel(table_hbm, idx_hbm, sig_hbm, out_hbm,
                   sig_v, idx_v, rows_v, g0, g1, o0, o1):
        cid = lax.axis_index("c")
        sid = lax.axis_index("s")
        wid = sid * _NC + cid
        base = wid * per_w
        gsems = (g0, g1)
        osems = (o0, o1)
        pltpu.sync_copy(sig_hbm, sig_v)
        pltpu.sync_copy(idx_hbm.at[pl.ds(base, per_w)], idx_v)

        def gather_desc(i, b, p, make_only):
            mk = pltpu.make_async_copy if make_only else pltpu.async_copy
            return mk(
                table_hbm.at[idx_v.at[pl.ds(i * _CHUNK + p * _GPIECE, _GPIECE)]],
                rows_v.at[b].at[pl.ds(p * _GPIECE, _GPIECE)],
                gsems[b],
            )

        def out_desc(i, b, make_only):
            mk = pltpu.make_async_copy if make_only else pltpu.async_copy
            return mk(rows_v.at[b], out_hbm.at[pl.ds(base + i * _CHUNK, _CHUNK)],
                      osems[b])

        def compute(b):
            @pl.loop(0, half, unroll=2)
            def _row(j):
                for d in range(dim // _LANES):
                    sl = pl.ds(d * _LANES, _LANES)
                    s = sig_v[j, sl]
                    rows_v[b, j, sl] = rows_v[b, j, sl] * scale + s
                    rows_v[b, half + j, sl] = rows_v[b, half + j, sl] * scale + s

        for p in range(n_pieces):
            gather_desc(0, 0, p, False)

        @pl.loop(0, n_chunks, step=2)
        def _chunk(i):
            for b in range(2):
                cur = i + b
                other = 1 - b

                @pl.when(cur + 1 < n_chunks)
                def _fire_next():
                    @pl.when(cur >= 1)
                    def _wait_out():
                        out_desc(cur - 1, other, True).wait()
                    for p in range(n_pieces):
                        gather_desc(cur + 1, other, p, False)

                for p in range(n_pieces):
                    gather_desc(cur, b, p, True).wait()
                compute(b)
                out_desc(cur, b, False)

        out_desc(n_chunks - 2, (n_chunks - 2) % 2, True).wait()
        out_desc(n_chunks - 1, (n_chunks - 1) % 2, True).wait()

    return emb_kernel


def kernel(x, table):
    b, seq_len = x.shape
    num_emb, dim = table.shape
    scale = float(dim) ** 0.5
    sig = _pos_signal(seq_len, dim)
    xf = x.reshape(-1)
    sc = _make_sc_kernel(b * seq_len, seq_len, dim, scale)
    out = sc(table, xf, sig)
    return out.reshape(b, seq_len, dim)
